# Initial kernel scaffold; baseline (speedup 1.0000x reference)
import jax
import jax.numpy as jnp
from jax.experimental import pallas as pl


def kernel(x, edge_index, edge_label_index, edge_attr, W1, b1, W2, b2, mlp_w1, mlp_b1, mlp_w2, mlp_b2):
    raise NotImplementedError("write your pallas kernel here")


# speedup vs baseline: 4.2626x; 4.2626x over previous
import functools

import jax
import jax.numpy as jnp
from jax import lax
from jax.experimental import pallas as pl
from jax.experimental.pallas import tpu as pltpu
from jax.experimental.pallas import tpu_sc as plsc

N_NODES = 10000
N_EDGES = 160000
D_IN = 256
HID = 256
D_EDGE = 16

NP = 10240
SENT = 10200
EPAD = 163840
BATCH = 128
NC = 2
NS = 16

_mesh = plsc.VectorSubcoreMesh(core_axis_name="c", subcore_axis_name="s")


@functools.partial(
    pl.kernel,
    out_type=jax.ShapeDtypeStruct((NC, NP), jnp.float32),
    mesh=_mesh,
    scratch_types=[
        pltpu.VMEM((EPAD // 32 // BATCH, BATCH), jnp.int32),
        pltpu.VMEM((BATCH,), jnp.float32),
        pltpu.VMEM_SHARED((NP,), jnp.float32),
        pltpu.SemaphoreType.DMA,
    ],
)
def _deg_kernel(dst_hbm, zeros_hbm, out_hbm, idx_v, ones_v, acc, sem):
    c = lax.axis_index("c")
    s = lax.axis_index("s")
    wid = s * NC + c
    nrow = NP // NS
    for i in range(BATCH // 16):
        ones_v[pl.ds(i * 16, 16)] = jnp.ones((16,), jnp.float32)
    pltpu.sync_copy(zeros_hbm.at[pl.ds(s * nrow, nrow)], acc.at[pl.ds(s * nrow, nrow)])
    pltpu.sync_copy(dst_hbm.at[wid], idx_v)
    plsc.subcore_barrier()

    def body(j, carry):
        pltpu.sync_copy(ones_v, acc.at[idx_v.at[j]], add=True)
        return carry

    lax.fori_loop(0, EPAD // 32 // BATCH, body, 0)
    plsc.subcore_barrier()
    pltpu.sync_copy(acc.at[pl.ds(s * nrow, nrow)], out_hbm.at[c, pl.ds(s * nrow, nrow)])


@functools.partial(
    pl.kernel,
    out_type=(
        jax.ShapeDtypeStruct((NP, 128), jnp.float32),
        jax.ShapeDtypeStruct((NP, 128), jnp.float32),
    ),
    mesh=_mesh,
    scratch_types=[
        pltpu.VMEM((EPAD // 16 // BATCH, BATCH), jnp.int32),
        pltpu.VMEM((EPAD // 16 // BATCH, BATCH), jnp.int32),
        pltpu.VMEM((BATCH, 128), jnp.float32),
        pltpu.VMEM_SHARED((NP, 128), jnp.float32),
        pltpu.SemaphoreType.DMA,
    ],
)
def _conv_kernel(y0_hbm, y1_hbm, src_hbm, dst_hbm, zeros_hbm, out0_hbm, out1_hbm,
                 src_v, dst_v, buf, acc, sem):
    c = lax.axis_index("c")
    s = lax.axis_index("s")
    nrow = NP // NS
    nb = EPAD // 16 // BATCH
    pltpu.sync_copy(zeros_hbm.at[pl.ds(s * nrow, nrow)], acc.at[pl.ds(s * nrow, nrow)])
    pltpu.sync_copy(src_hbm.at[s], src_v)
    pltpu.sync_copy(dst_hbm.at[s], dst_v)
    plsc.subcore_barrier()

    def run(table_hbm, out_hbm):
        def body(j, carry):
            pltpu.async_copy(table_hbm.at[src_v.at[j]], buf, sem).wait()
            pltpu.sync_copy(buf, acc.at[dst_v.at[j]], add=True)
            return carry

        lax.fori_loop(0, nb, body, 0)
        plsc.subcore_barrier()
        pltpu.sync_copy(acc.at[pl.ds(s * nrow, nrow)], out_hbm.at[pl.ds(s * nrow, nrow)])

    @pl.when(c == 0)
    def _():
        run(y0_hbm, out0_hbm)

    @pl.when(c == 1)
    def _():
        run(y1_hbm, out1_hbm)


@functools.partial(
    pl.kernel,
    out_type=(
        jax.ShapeDtypeStruct((EPAD, HID), jnp.float32),
        jax.ShapeDtypeStruct((EPAD, HID), jnp.float32),
    ),
    mesh=_mesh,
    scratch_types=[
        pltpu.VMEM((EPAD // 32 // BATCH, BATCH), jnp.int32),
        pltpu.VMEM((EPAD // 32 // BATCH, BATCH), jnp.int32),
        pltpu.VMEM((BATCH, HID), jnp.float32),
        pltpu.VMEM((BATCH, HID), jnp.float32),
        pltpu.SemaphoreType.DMA,
        pltpu.SemaphoreType.DMA,
    ],
)
def _decode_kernel(a_hbm, b_hbm, u_hbm, v_hbm, ga_hbm, gb_hbm,
                   u_v, v_v, bufa, bufb, sem1, sem2):
    c = lax.axis_index("c")
    s = lax.axis_index("s")
    wid = s * NC + c
    nb = EPAD // 32 // BATCH
    pltpu.sync_copy(u_hbm.at[wid], u_v)
    pltpu.sync_copy(v_hbm.at[wid], v_v)

    def body(j, carry):
        ca = pltpu.async_copy(a_hbm.at[u_v.at[j]], bufa, sem1)
        cb = pltpu.async_copy(b_hbm.at[v_v.at[j]], bufb, sem2)
        ca.wait()
        pltpu.sync_copy(bufa, ga_hbm.at[pl.ds(wid * (nb * BATCH) + j * BATCH, BATCH)])
        cb.wait()
        pltpu.sync_copy(bufb, gb_hbm.at[pl.ds(wid * (nb * BATCH) + j * BATCH, BATCH)])
        return carry

    lax.fori_loop(0, nb, body, 0)


def _k1_body(x_ref, w_ref, deg_ref, y0_ref, y1_ref):
    d = deg_ref[...]
    dis = lax.rsqrt(d[0] + d[1] + 1.0)
    yw = dis[:, None] * jnp.dot(x_ref[...], w_ref[...],
                                preferred_element_type=jnp.float32)
    y0_ref[...] = yw[:, :128]
    y1_ref[...] = yw[:, 128:]


def _k2_body(s0_ref, s1_ref, y0_ref, y1_ref, deg_ref, b1_ref, w2_ref,
             o0_ref, o1_ref):
    d = deg_ref[...]
    dis = lax.rsqrt(d[0] + d[1] + 1.0)
    sfull = jnp.concatenate([s0_ref[...] + y0_ref[...],
                             s1_ref[...] + y1_ref[...]], axis=1)
    h = jnp.maximum(dis[:, None] * sfull + b1_ref[...], 0.0)
    hw2 = dis[:, None] * jnp.dot(h, w2_ref[...],
                                 preferred_element_type=jnp.float32)
    o0_ref[...] = hw2[:, :128]
    o1_ref[...] = hw2[:, 128:]


def _k3_body(s0_ref, s1_ref, y0_ref, y1_ref, deg_ref, b2_ref, wu_ref, wv_ref,
             a_ref, b_ref):
    d = deg_ref[...]
    dis = lax.rsqrt(d[0] + d[1] + 1.0)
    sfull = jnp.concatenate([s0_ref[...] + y0_ref[...],
                             s1_ref[...] + y1_ref[...]], axis=1)
    z = dis[:, None] * sfull + b2_ref[...]
    a_ref[...] = jnp.dot(z, wu_ref[...], preferred_element_type=jnp.float32)
    b_ref[...] = jnp.dot(z, wv_ref[...], preferred_element_type=jnp.float32)


def _k4_body(ga_ref, gb_ref, ea_ref, we_ref, mb1_ref, w2_ref, mb2_ref, o_ref):
    hdn = jnp.maximum(
        ga_ref[...] + gb_ref[...]
        + jnp.dot(ea_ref[...], we_ref[...], preferred_element_type=jnp.float32)
        + mb1_ref[...], 0.0)
    o_ref[...] = jnp.dot(hdn, w2_ref[...],
                         preferred_element_type=jnp.float32) + mb2_ref[...]


def kernel(x, edge_index, edge_label_index, edge_attr,
           W1, b1, W2, b2, mlp_w1, mlp_b1, mlp_w2, mlp_b2):
    f32 = jnp.float32
    x_pad = jnp.zeros((NP, D_IN), f32).at[:N_NODES].set(x)
    src = edge_index[0].astype(jnp.int32)
    dst = edge_index[1].astype(jnp.int32)
    u = edge_label_index[0].astype(jnp.int32)
    v = edge_label_index[1].astype(jnp.int32)
    epad = jnp.full((EPAD - N_EDGES,), SENT, jnp.int32)
    src16 = jnp.concatenate([src, epad]).reshape(NS, EPAD // NS // BATCH, BATCH)
    dst16 = jnp.concatenate([dst, epad]).reshape(NS, EPAD // NS // BATCH, BATCH)
    dst32 = jnp.concatenate([dst, epad]).reshape(32, EPAD // 32 // BATCH, BATCH)
    u32 = jnp.concatenate([u, epad]).reshape(32, EPAD // 32 // BATCH, BATCH)
    v32 = jnp.concatenate([v, epad]).reshape(32, EPAD // 32 // BATCH, BATCH)
    ea_pad = jnp.zeros((EPAD, D_EDGE), f32).at[:N_EDGES].set(edge_attr)
    zeros_n = jnp.zeros((NP, 128), f32)
    zeros_1 = jnp.zeros((NP,), f32)
    wu = mlp_w1[:HID]
    wv = mlp_w1[HID:2 * HID]
    we = mlp_w1[2 * HID:]
    b1r = b1.reshape(1, HID)
    b2r = b2.reshape(1, HID)
    mb1 = mlp_b1.reshape(1, HID)
    mb2 = mlp_b2.reshape(1, 1)

    deg = _deg_kernel(dst32, zeros_1)

    bm = 256
    grid = (NP // bm,)
    y0, y1 = pl.pallas_call(
        _k1_body,
        grid=grid,
        in_specs=[
            pl.BlockSpec((bm, D_IN), lambda i: (i, 0)),
            pl.BlockSpec((D_IN, HID), lambda i: (0, 0)),
            pl.BlockSpec((NC, bm), lambda i: (0, i)),
        ],
        out_specs=[
            pl.BlockSpec((bm, 128), lambda i: (i, 0)),
            pl.BlockSpec((bm, 128), lambda i: (i, 0)),
        ],
        out_shape=[
            jax.ShapeDtypeStruct((NP, 128), f32),
            jax.ShapeDtypeStruct((NP, 128), f32),
        ],
    )(x_pad, W1, deg)

    s0, s1 = _conv_kernel(y0, y1, src16, dst16, zeros_n)

    y20, y21 = pl.pallas_call(
        _k2_body,
        grid=grid,
        in_specs=[
            pl.BlockSpec((bm, 128), lambda i: (i, 0)),
            pl.BlockSpec((bm, 128), lambda i: (i, 0)),
            pl.BlockSpec((bm, 128), lambda i: (i, 0)),
            pl.BlockSpec((bm, 128), lambda i: (i, 0)),
            pl.BlockSpec((NC, bm), lambda i: (0, i)),
            pl.BlockSpec((1, HID), lambda i: (0, 0)),
            pl.BlockSpec((HID, HID), lambda i: (0, 0)),
        ],
        out_specs=[
            pl.BlockSpec((bm, 128), lambda i: (i, 0)),
            pl.BlockSpec((bm, 128), lambda i: (i, 0)),
        ],
        out_shape=[
            jax.ShapeDtypeStruct((NP, 128), f32),
            jax.ShapeDtypeStruct((NP, 128), f32),
        ],
    )(s0, s1, y0, y1, deg, b1r, W2)

    t0, t1 = _conv_kernel(y20, y21, src16, dst16, zeros_n)

    A, B = pl.pallas_call(
        _k3_body,
        grid=grid,
        in_specs=[
            pl.BlockSpec((bm, 128), lambda i: (i, 0)),
            pl.BlockSpec((bm, 128), lambda i: (i, 0)),
            pl.BlockSpec((bm, 128), lambda i: (i, 0)),
            pl.BlockSpec((bm, 128), lambda i: (i, 0)),
            pl.BlockSpec((NC, bm), lambda i: (0, i)),
            pl.BlockSpec((1, HID), lambda i: (0, 0)),
            pl.BlockSpec((HID, HID), lambda i: (0, 0)),
            pl.BlockSpec((HID, HID), lambda i: (0, 0)),
        ],
        out_specs=[
            pl.BlockSpec((bm, HID), lambda i: (i, 0)),
            pl.BlockSpec((bm, HID), lambda i: (i, 0)),
        ],
        out_shape=[
            jax.ShapeDtypeStruct((NP, HID), f32),
            jax.ShapeDtypeStruct((NP, HID), f32),
        ],
    )(t0, t1, y20, y21, deg, b2r, wu, wv)

    GA, GB = _decode_kernel(A, B, u32, v32)

    bm4 = 2048
    out = pl.pallas_call(
        _k4_body,
        grid=(EPAD // bm4,),
        in_specs=[
            pl.BlockSpec((bm4, HID), lambda i: (i, 0)),
            pl.BlockSpec((bm4, HID), lambda i: (i, 0)),
            pl.BlockSpec((bm4, D_EDGE), lambda i: (i, 0)),
            pl.BlockSpec((D_EDGE, HID), lambda i: (0, 0)),
            pl.BlockSpec((1, HID), lambda i: (0, 0)),
            pl.BlockSpec((HID, 1), lambda i: (0, 0)),
            pl.BlockSpec((1, 1), lambda i: (0, 0)),
        ],
        out_specs=pl.BlockSpec((bm4, 1), lambda i: (i, 0)),
        out_shape=jax.ShapeDtypeStruct((EPAD, 1), f32),
    )(GA, GB, ea_pad, we, mb1, mlp_w2, mb2)

    return out[:N_EDGES].reshape(-1)


# speedup vs baseline: 5.0590x; 1.1868x over previous
import functools

import jax
import jax.numpy as jnp
from jax import lax
from jax.experimental import pallas as pl
from jax.experimental.pallas import tpu as pltpu
from jax.experimental.pallas import tpu_sc as plsc

N_NODES = 10000
N_EDGES = 160000
D_IN = 256
HID = 256
D_EDGE = 16

NP = 10240
SENT = 10200
EPAD = 163840
BATCH = 128
NC = 2
NS = 16
NBT = EPAD // NS // BATCH
NPH = NBT // 2

_mesh = plsc.VectorSubcoreMesh(core_axis_name="c", subcore_axis_name="s")


@functools.partial(
    pl.kernel,
    out_type=jax.ShapeDtypeStruct((NC, NP), jnp.float32),
    mesh=_mesh,
    scratch_types=[
        pltpu.VMEM((EPAD // 32 // BATCH, BATCH), jnp.int32),
        pltpu.VMEM((BATCH,), jnp.float32),
        pltpu.VMEM_SHARED((NP,), jnp.float32),
        pltpu.SemaphoreType.DMA,
    ],
)
def _deg_kernel(dst_hbm, zeros_hbm, out_hbm, idx_v, ones_v, acc, sem):
    c = lax.axis_index("c")
    s = lax.axis_index("s")
    wid = s * NC + c
    nrow = NP // NS
    for i in range(BATCH // 16):
        ones_v[pl.ds(i * 16, 16)] = jnp.ones((16,), jnp.float32)
    pltpu.sync_copy(zeros_hbm.at[pl.ds(s * nrow, nrow)], acc.at[pl.ds(s * nrow, nrow)])
    pltpu.sync_copy(dst_hbm.at[wid], idx_v)
    plsc.subcore_barrier()

    def body(j, carry):
        pltpu.sync_copy(ones_v, acc.at[idx_v.at[j]], add=True)
        return carry

    lax.fori_loop(0, EPAD // 32 // BATCH, body, 0)
    plsc.subcore_barrier()
    pltpu.sync_copy(acc.at[pl.ds(s * nrow, nrow)], out_hbm.at[c, pl.ds(s * nrow, nrow)])


@functools.partial(
    pl.kernel,
    out_type=(
        jax.ShapeDtypeStruct((NP, 128), jnp.float32),
        jax.ShapeDtypeStruct((NP, 128), jnp.float32),
    ),
    mesh=_mesh,
    scratch_types=[
        pltpu.VMEM((NPH, BATCH), jnp.int32),
        pltpu.VMEM((NPH, BATCH), jnp.int32),
        pltpu.VMEM((BATCH, 128), jnp.float32),
        pltpu.VMEM((BATCH, 128), jnp.float32),
        pltpu.VMEM_SHARED((NP, 128), jnp.float32),
        pltpu.SemaphoreType.DMA,
        pltpu.SemaphoreType.DMA,
    ],
)
def _conv_kernel(y0_hbm, y1_hbm, src_hbm, dst_hbm, zeros_hbm, out0_hbm, out1_hbm,
                 src_v, dst_v, buf0, buf1, acc, sem0, sem1):
    c = lax.axis_index("c")
    s = lax.axis_index("s")
    nrow = NP // NS
    UN = 8
    pltpu.sync_copy(zeros_hbm.at[pl.ds(s * nrow, nrow)], acc.at[pl.ds(s * nrow, nrow)])
    plsc.subcore_barrier()

    def run(table_hbm, out_hbm):
        bufs = (buf0, buf1)
        sems = (sem0, sem1)

        def issue(j, b):
            pltpu.async_copy(table_hbm.at[src_v.at[j]], bufs[b], sems[b])

        def wait(b):
            pltpu.make_async_copy(table_hbm.at[pl.ds(0, BATCH)], bufs[b], sems[b]).wait()

        for ph in range(2):
            pltpu.sync_copy(src_hbm.at[s, pl.ds(ph * NPH, NPH)], src_v)
            pltpu.sync_copy(dst_hbm.at[s, pl.ds(ph * NPH, NPH)], dst_v)
            issue(0, 0)

            def body(i, carry):
                base = i * UN
                for k in range(UN):
                    cur = k & 1
                    nxt = 1 - cur
                    nj = base + k + 1

                    @pl.when(nj < NPH)
                    def _():
                        issue(nj, nxt)

                    wait(cur)
                    pltpu.sync_copy(bufs[cur], acc.at[dst_v.at[base + k]], add=True)
                return carry

            lax.fori_loop(0, NPH // UN, body, 0)
        plsc.subcore_barrier()
        pltpu.sync_copy(acc.at[pl.ds(s * nrow, nrow)], out_hbm.at[pl.ds(s * nrow, nrow)])

    @pl.when(c == 0)
    def _():
        run(y0_hbm, out0_hbm)

    @pl.when(c == 1)
    def _():
        run(y1_hbm, out1_hbm)


BDEC = 64


@functools.partial(
    pl.kernel,
    out_type=jax.ShapeDtypeStruct((EPAD, HID), jnp.float32),
    mesh=_mesh,
    scratch_types=[
        pltpu.VMEM((EPAD // 32 // BDEC, BDEC), jnp.int32),
        pltpu.VMEM((EPAD // 32 // BDEC, BDEC), jnp.int32),
        pltpu.VMEM((BDEC, HID), jnp.float32),
        pltpu.VMEM((BDEC, HID), jnp.float32),
        pltpu.VMEM((BDEC, HID), jnp.float32),
        pltpu.VMEM((BDEC, HID), jnp.float32),
        pltpu.SemaphoreType.DMA,
        pltpu.SemaphoreType.DMA,
        pltpu.SemaphoreType.DMA,
        pltpu.SemaphoreType.DMA,
    ],
)
def _decode_kernel(a_hbm, b_hbm, u_hbm, v_hbm, g_hbm,
                   u_v, v_v, a0, a1, b0, b1, sa0, sa1, sb0, sb1):
    c = lax.axis_index("c")
    s = lax.axis_index("s")
    wid = s * NC + c
    nb = EPAD // 32 // BDEC
    UN = 8
    abufs = (a0, a1)
    bbufs = (b0, b1)
    asems = (sa0, sa1)
    bsems = (sb0, sb1)
    pltpu.sync_copy(u_hbm.at[wid], u_v)
    pltpu.sync_copy(v_hbm.at[wid], v_v)

    def issue(j, b):
        pltpu.async_copy(a_hbm.at[u_v.at[j]], abufs[b], asems[b])
        pltpu.async_copy(b_hbm.at[v_v.at[j]], bbufs[b], bsems[b])

    def wait(b):
        pltpu.make_async_copy(a_hbm.at[pl.ds(0, BDEC)], abufs[b], asems[b]).wait()
        pltpu.make_async_copy(b_hbm.at[pl.ds(0, BDEC)], bbufs[b], bsems[b]).wait()

    issue(0, 0)

    def body(i, carry):
        base = i * UN
        for k in range(UN):
            cur = k & 1
            nxt = 1 - cur
            nj = base + k + 1

            @pl.when(nj < nb)
            def _():
                issue(nj, nxt)

            wait(cur)
            av = abufs[cur]
            bv = bbufs[cur]

            def add_row(r, carry2):
                for cc in range(HID // 16):
                    sl = pl.ds(cc * 16, 16)
                    av[r, sl] = av[r, sl] + bv[r, sl]
                return carry2

            lax.fori_loop(0, BDEC, add_row, 0)
            pltpu.sync_copy(av, g_hbm.at[pl.ds(wid * (nb * BDEC) + (base + k) * BDEC, BDEC)])
        return carry

    lax.fori_loop(0, nb // UN, body, 0)


def _k1_body(x_ref, w_ref, deg_ref, y0_ref, y1_ref):
    d = deg_ref[...]
    dis = lax.rsqrt(d[0] + d[1] + 1.0)
    yw = dis[:, None] * jnp.dot(x_ref[...], w_ref[...],
                                preferred_element_type=jnp.float32)
    y0_ref[...] = yw[:, :128]
    y1_ref[...] = yw[:, 128:]


def _k2_body(s0_ref, s1_ref, y0_ref, y1_ref, deg_ref, b1_ref, w2_ref,
             o0_ref, o1_ref):
    d = deg_ref[...]
    dis = lax.rsqrt(d[0] + d[1] + 1.0)
    sfull = jnp.concatenate([s0_ref[...] + y0_ref[...],
                             s1_ref[...] + y1_ref[...]], axis=1)
    h = jnp.maximum(dis[:, None] * sfull + b1_ref[...], 0.0)
    hw2 = dis[:, None] * jnp.dot(h, w2_ref[...],
                                 preferred_element_type=jnp.float32)
    o0_ref[...] = hw2[:, :128]
    o1_ref[...] = hw2[:, 128:]


def _k3_body(s0_ref, s1_ref, y0_ref, y1_ref, deg_ref, b2_ref, wu_ref, wv_ref,
             a_ref, b_ref):
    d = deg_ref[...]
    dis = lax.rsqrt(d[0] + d[1] + 1.0)
    sfull = jnp.concatenate([s0_ref[...] + y0_ref[...],
                             s1_ref[...] + y1_ref[...]], axis=1)
    z = dis[:, None] * sfull + b2_ref[...]
    a_ref[...] = jnp.dot(z, wu_ref[...], preferred_element_type=jnp.float32)
    b_ref[...] = jnp.dot(z, wv_ref[...], preferred_element_type=jnp.float32)


def _k4_body(g_ref, ea_ref, we_ref, mb1_ref, w2_ref, mb2_ref, o_ref):
    hdn = jnp.maximum(
        g_ref[...]
        + jnp.dot(ea_ref[...], we_ref[...], preferred_element_type=jnp.float32)
        + mb1_ref[...], 0.0)
    o_ref[...] = jnp.dot(hdn, w2_ref[...],
                         preferred_element_type=jnp.float32) + mb2_ref[...]


def kernel(x, edge_index, edge_label_index, edge_attr,
           W1, b1, W2, b2, mlp_w1, mlp_b1, mlp_w2, mlp_b2):
    f32 = jnp.float32
    x_pad = jnp.zeros((NP, D_IN), f32).at[:N_NODES].set(x)
    src = edge_index[0].astype(jnp.int32)
    dst = edge_index[1].astype(jnp.int32)
    u = edge_label_index[0].astype(jnp.int32)
    v = edge_label_index[1].astype(jnp.int32)
    epad = jnp.full((EPAD - N_EDGES,), SENT, jnp.int32)
    src16 = jnp.concatenate([src, epad]).reshape(NS, NBT, BATCH)
    dst16 = jnp.concatenate([dst, epad]).reshape(NS, NBT, BATCH)
    dst32 = jnp.concatenate([dst, epad]).reshape(32, EPAD // 32 // BATCH, BATCH)
    u32 = jnp.concatenate([u, epad]).reshape(32, EPAD // 32 // BDEC, BDEC)
    v32 = jnp.concatenate([v, epad]).reshape(32, EPAD // 32 // BDEC, BDEC)
    ea_pad = jnp.zeros((EPAD, D_EDGE), f32).at[:N_EDGES].set(edge_attr)
    zeros_n = jnp.zeros((NP, 128), f32)
    zeros_1 = jnp.zeros((NP,), f32)
    wu = mlp_w1[:HID]
    wv = mlp_w1[HID:2 * HID]
    we = mlp_w1[2 * HID:]
    b1r = b1.reshape(1, HID)
    b2r = b2.reshape(1, HID)
    mb1 = mlp_b1.reshape(1, HID)
    mb2 = mlp_b2.reshape(1, 1)

    deg = _deg_kernel(dst32, zeros_1)

    bm = 256
    grid = (NP // bm,)
    y0, y1 = pl.pallas_call(
        _k1_body,
        grid=grid,
        in_specs=[
            pl.BlockSpec((bm, D_IN), lambda i: (i, 0)),
            pl.BlockSpec((D_IN, HID), lambda i: (0, 0)),
            pl.BlockSpec((NC, bm), lambda i: (0, i)),
        ],
        out_specs=[
            pl.BlockSpec((bm, 128), lambda i: (i, 0)),
            pl.BlockSpec((bm, 128), lambda i: (i, 0)),
        ],
        out_shape=[
            jax.ShapeDtypeStruct((NP, 128), f32),
            jax.ShapeDtypeStruct((NP, 128), f32),
        ],
    )(x_pad, W1, deg)

    s0, s1 = _conv_kernel(y0, y1, src16, dst16, zeros_n)

    y20, y21 = pl.pallas_call(
        _k2_body,
        grid=grid,
        in_specs=[
            pl.BlockSpec((bm, 128), lambda i: (i, 0)),
            pl.BlockSpec((bm, 128), lambda i: (i, 0)),
            pl.BlockSpec((bm, 128), lambda i: (i, 0)),
            pl.BlockSpec((bm, 128), lambda i: (i, 0)),
            pl.BlockSpec((NC, bm), lambda i: (0, i)),
            pl.BlockSpec((1, HID), lambda i: (0, 0)),
            pl.BlockSpec((HID, HID), lambda i: (0, 0)),
        ],
        out_specs=[
            pl.BlockSpec((bm, 128), lambda i: (i, 0)),
            pl.BlockSpec((bm, 128), lambda i: (i, 0)),
        ],
        out_shape=[
            jax.ShapeDtypeStruct((NP, 128), f32),
            jax.ShapeDtypeStruct((NP, 128), f32),
        ],
    )(s0, s1, y0, y1, deg, b1r, W2)

    t0, t1 = _conv_kernel(y20, y21, src16, dst16, zeros_n)

    A, B = pl.pallas_call(
        _k3_body,
        grid=grid,
        in_specs=[
            pl.BlockSpec((bm, 128), lambda i: (i, 0)),
            pl.BlockSpec((bm, 128), lambda i: (i, 0)),
            pl.BlockSpec((bm, 128), lambda i: (i, 0)),
            pl.BlockSpec((bm, 128), lambda i: (i, 0)),
            pl.BlockSpec((NC, bm), lambda i: (0, i)),
            pl.BlockSpec((1, HID), lambda i: (0, 0)),
            pl.BlockSpec((HID, HID), lambda i: (0, 0)),
            pl.BlockSpec((HID, HID), lambda i: (0, 0)),
        ],
        out_specs=[
            pl.BlockSpec((bm, HID), lambda i: (i, 0)),
            pl.BlockSpec((bm, HID), lambda i: (i, 0)),
        ],
        out_shape=[
            jax.ShapeDtypeStruct((NP, HID), f32),
            jax.ShapeDtypeStruct((NP, HID), f32),
        ],
    )(t0, t1, y20, y21, deg, b2r, wu, wv)

    G = _decode_kernel(A, B, u32, v32)

    bm4 = 2048
    out = pl.pallas_call(
        _k4_body,
        grid=(EPAD // bm4,),
        in_specs=[
            pl.BlockSpec((bm4, HID), lambda i: (i, 0)),
            pl.BlockSpec((bm4, D_EDGE), lambda i: (i, 0)),
            pl.BlockSpec((D_EDGE, HID), lambda i: (0, 0)),
            pl.BlockSpec((1, HID), lambda i: (0, 0)),
            pl.BlockSpec((HID, 1), lambda i: (0, 0)),
            pl.BlockSpec((1, 1), lambda i: (0, 0)),
        ],
        out_specs=pl.BlockSpec((bm4, 1), lambda i: (i, 0)),
        out_shape=jax.ShapeDtypeStruct((EPAD, 1), f32),
    )(G, ea_pad, we, mb1, mlp_w2, mb2)

    return out[:N_EDGES].reshape(-1)


# speedup vs baseline: 5.3208x; 1.0518x over previous
import functools

import jax
import jax.numpy as jnp
from jax import lax
from jax.experimental import pallas as pl
from jax.experimental.pallas import tpu as pltpu
from jax.experimental.pallas import tpu_sc as plsc

N_NODES = 10000
N_EDGES = 160000
D_IN = 256
HID = 256
D_EDGE = 16

NP = 10240
SENT = 10200
EPAD = 163840
BATCH = 128
NC = 2
NS = 16
NBT = EPAD // NS // BATCH
NPH = NBT // 2

_mesh = plsc.VectorSubcoreMesh(core_axis_name="c", subcore_axis_name="s")


@functools.partial(
    pl.kernel,
    out_type=jax.ShapeDtypeStruct((NC, NP), jnp.float32),
    mesh=_mesh,
    scratch_types=[
        pltpu.VMEM((EPAD // 32 // BATCH, BATCH), jnp.int32),
        pltpu.VMEM((BATCH,), jnp.float32),
        pltpu.VMEM_SHARED((NP,), jnp.float32),
        pltpu.SemaphoreType.DMA,
    ],
)
def _deg_kernel(dst_hbm, zeros_hbm, out_hbm, idx_v, ones_v, acc, sem):
    c = lax.axis_index("c")
    s = lax.axis_index("s")
    wid = s * NC + c
    nrow = NP // NS
    for i in range(BATCH // 16):
        ones_v[pl.ds(i * 16, 16)] = jnp.ones((16,), jnp.float32)
    pltpu.sync_copy(zeros_hbm.at[pl.ds(s * nrow, nrow)], acc.at[pl.ds(s * nrow, nrow)])
    pltpu.sync_copy(dst_hbm.at[wid], idx_v)
    plsc.subcore_barrier()

    def body(j, carry):
        pltpu.sync_copy(ones_v, acc.at[idx_v.at[j]], add=True)
        return carry

    lax.fori_loop(0, EPAD // 32 // BATCH, body, 0)
    plsc.subcore_barrier()
    pltpu.sync_copy(acc.at[pl.ds(s * nrow, nrow)], out_hbm.at[c, pl.ds(s * nrow, nrow)])


@functools.partial(
    pl.kernel,
    out_type=(
        jax.ShapeDtypeStruct((NP, 128), jnp.float32),
        jax.ShapeDtypeStruct((NP, 128), jnp.float32),
    ),
    mesh=_mesh,
    scratch_types=[
        pltpu.VMEM((NPH, BATCH), jnp.int32),
        pltpu.VMEM((NPH, BATCH), jnp.int32),
        pltpu.VMEM((BATCH, 128), jnp.float32),
        pltpu.VMEM((BATCH, 128), jnp.float32),
        pltpu.VMEM_SHARED((NP, 128), jnp.float32),
        pltpu.SemaphoreType.DMA,
        pltpu.SemaphoreType.DMA,
    ],
)
def _conv_kernel(y0_hbm, y1_hbm, src_hbm, dst_hbm, zeros_hbm, out0_hbm, out1_hbm,
                 src_v, dst_v, buf0, buf1, acc, sem0, sem1):
    c = lax.axis_index("c")
    s = lax.axis_index("s")
    nrow = NP // NS
    UN = 8
    pltpu.sync_copy(zeros_hbm.at[pl.ds(s * nrow, nrow)], acc.at[pl.ds(s * nrow, nrow)])
    plsc.subcore_barrier()

    def run(table_hbm, out_hbm):
        bufs = (buf0, buf1)
        sems = (sem0, sem1)

        def issue(j, b):
            pltpu.async_copy(table_hbm.at[src_v.at[j]], bufs[b], sems[b])

        def wait(b):
            pltpu.make_async_copy(table_hbm.at[pl.ds(0, BATCH)], bufs[b], sems[b]).wait()

        for ph in range(2):
            pltpu.sync_copy(src_hbm.at[s, pl.ds(ph * NPH, NPH)], src_v)
            pltpu.sync_copy(dst_hbm.at[s, pl.ds(ph * NPH, NPH)], dst_v)
            issue(0, 0)

            def body(i, carry):
                base = i * UN
                for k in range(UN):
                    cur = k & 1
                    nxt = 1 - cur
                    nj = base + k + 1

                    @pl.when(nj < NPH)
                    def _():
                        issue(nj, nxt)

                    wait(cur)
                    pltpu.sync_copy(bufs[cur], acc.at[dst_v.at[base + k]], add=True)
                return carry

            lax.fori_loop(0, NPH // UN, body, 0)
        plsc.subcore_barrier()
        pltpu.sync_copy(acc.at[pl.ds(s * nrow, nrow)], out_hbm.at[pl.ds(s * nrow, nrow)])

    @pl.when(c == 0)
    def _():
        run(y0_hbm, out0_hbm)

    @pl.when(c == 1)
    def _():
        run(y1_hbm, out1_hbm)


BDEC = 128


@functools.partial(
    pl.kernel,
    out_type=(
        jax.ShapeDtypeStruct((EPAD, 128), jnp.int32),
        jax.ShapeDtypeStruct((EPAD, 128), jnp.int32),
    ),
    mesh=_mesh,
    scratch_types=[
        pltpu.VMEM((EPAD // 32 // BDEC, BDEC), jnp.int32),
        pltpu.VMEM((EPAD // 32 // BDEC, BDEC), jnp.int32),
        pltpu.VMEM((BDEC, 128), jnp.int32),
        pltpu.VMEM((BDEC, 128), jnp.int32),
        pltpu.VMEM((BDEC, 128), jnp.int32),
        pltpu.VMEM((BDEC, 128), jnp.int32),
        pltpu.SemaphoreType.DMA,
        pltpu.SemaphoreType.DMA,
        pltpu.SemaphoreType.DMA,
        pltpu.SemaphoreType.DMA,
        pltpu.SemaphoreType.DMA,
        pltpu.SemaphoreType.DMA,
        pltpu.SemaphoreType.DMA,
        pltpu.SemaphoreType.DMA,
    ],
)
def _decode_kernel(a_hbm, b_hbm, u_hbm, v_hbm, ga_hbm, gb_hbm,
                   u_v, v_v, a0, a1, b0, b1,
                   sa0, sa1, sb0, sb1, oa0, oa1, ob0, ob1):
    c = lax.axis_index("c")
    s = lax.axis_index("s")
    wid = s * NC + c
    nb = EPAD // 32 // BDEC
    UN = 8
    abufs = (a0, a1)
    bbufs = (b0, b1)
    asems = (sa0, sa1)
    bsems = (sb0, sb1)
    oasems = (oa0, oa1)
    obsems = (ob0, ob1)
    pltpu.sync_copy(u_hbm.at[wid], u_v)
    pltpu.sync_copy(v_hbm.at[wid], v_v)

    def issue(j, b):
        pltpu.async_copy(a_hbm.at[u_v.at[j]], abufs[b], asems[b])
        pltpu.async_copy(b_hbm.at[v_v.at[j]], bbufs[b], bsems[b])

    def wait(b):
        pltpu.make_async_copy(a_hbm.at[pl.ds(0, BDEC)], abufs[b], asems[b]).wait()
        pltpu.make_async_copy(b_hbm.at[pl.ds(0, BDEC)], bbufs[b], bsems[b]).wait()

    def drain_outs(b):
        pltpu.make_async_copy(a_hbm.at[pl.ds(0, BDEC)], abufs[b], oasems[b]).wait()
        pltpu.make_async_copy(a_hbm.at[pl.ds(0, BDEC)], bbufs[b], obsems[b]).wait()

    issue(0, 0)

    def body(i, carry):
        base = i * UN
        for k in range(UN):
            cur = k & 1
            nxt = 1 - cur
            nj = base + k + 1

            @pl.when((nj < nb) & (nj >= 2))
            def _():
                drain_outs(nxt)

            @pl.when(nj < nb)
            def _():
                issue(nj, nxt)

            wait(cur)
            off = wid * (nb * BDEC) + (base + k) * BDEC
            pltpu.async_copy(abufs[cur], ga_hbm.at[pl.ds(off, BDEC)], oasems[cur])
            pltpu.async_copy(bbufs[cur], gb_hbm.at[pl.ds(off, BDEC)], obsems[cur])
        return carry

    lax.fori_loop(0, nb // UN, body, 0)
    drain_outs(0)
    drain_outs(1)


def _k1_body(x_ref, w_ref, deg_ref, y0_ref, y1_ref):
    d = deg_ref[...]
    dis = lax.rsqrt(d[0] + d[1] + 1.0)
    yw = dis[:, None] * jnp.dot(x_ref[...], w_ref[...],
                                preferred_element_type=jnp.float32)
    y0_ref[...] = yw[:, :128]
    y1_ref[...] = yw[:, 128:]


def _k2_body(s0_ref, s1_ref, y0_ref, y1_ref, deg_ref, b1_ref, w2_ref,
             o0_ref, o1_ref):
    d = deg_ref[...]
    dis = lax.rsqrt(d[0] + d[1] + 1.0)
    sfull = jnp.concatenate([s0_ref[...] + y0_ref[...],
                             s1_ref[...] + y1_ref[...]], axis=1)
    h = jnp.maximum(dis[:, None] * sfull + b1_ref[...], 0.0)
    hw2 = dis[:, None] * jnp.dot(h, w2_ref[...],
                                 preferred_element_type=jnp.float32)
    o0_ref[...] = hw2[:, :128]
    o1_ref[...] = hw2[:, 128:]


def _k3_body(s0_ref, s1_ref, y0_ref, y1_ref, deg_ref, b2_ref, wu_ref, wv_ref,
             a_ref, b_ref):
    d = deg_ref[...]
    dis = lax.rsqrt(d[0] + d[1] + 1.0)
    sfull = jnp.concatenate([s0_ref[...] + y0_ref[...],
                             s1_ref[...] + y1_ref[...]], axis=1)
    z = dis[:, None] * sfull + b2_ref[...]
    A = jnp.dot(z, wu_ref[...], preferred_element_type=jnp.float32)
    B = jnp.dot(z, wv_ref[...], preferred_element_type=jnp.float32)

    def pack(m):
        lo = jax.lax.bitcast_convert_type(m[:, :128].astype(jnp.bfloat16),
                                          jnp.uint16).astype(jnp.uint32)
        hi = jax.lax.bitcast_convert_type(m[:, 128:].astype(jnp.bfloat16),
                                          jnp.uint16).astype(jnp.uint32)
        return (lo | (hi << 16)).astype(jnp.int32)

    a_ref[...] = pack(A)
    b_ref[...] = pack(B)


def _unpack(gw):
    lo = jax.lax.bitcast_convert_type(
        (gw & 0xFFFF).astype(jnp.uint16), jnp.bfloat16)
    hi = jax.lax.bitcast_convert_type(
        ((gw >> 16) & 0xFFFF).astype(jnp.uint16), jnp.bfloat16)
    return jnp.concatenate([lo, hi], axis=1).astype(jnp.float32)


def _k4_body(ga_ref, gb_ref, ea_ref, we_ref, mb1_ref, w2_ref, mb2_ref, o_ref):
    g = _unpack(ga_ref[...]) + _unpack(gb_ref[...])
    hdn = jnp.maximum(
        g
        + jnp.dot(ea_ref[...], we_ref[...], preferred_element_type=jnp.float32)
        + mb1_ref[...], 0.0)
    o_ref[...] = jnp.dot(hdn, w2_ref[...],
                         preferred_element_type=jnp.float32) + mb2_ref[...]


def kernel(x, edge_index, edge_label_index, edge_attr,
           W1, b1, W2, b2, mlp_w1, mlp_b1, mlp_w2, mlp_b2):
    f32 = jnp.float32
    x_pad = jnp.zeros((NP, D_IN), f32).at[:N_NODES].set(x)
    src = edge_index[0].astype(jnp.int32)
    dst = edge_index[1].astype(jnp.int32)
    u = edge_label_index[0].astype(jnp.int32)
    v = edge_label_index[1].astype(jnp.int32)
    epad = jnp.full((EPAD - N_EDGES,), SENT, jnp.int32)
    src16 = jnp.concatenate([src, epad]).reshape(NS, NBT, BATCH)
    dst16 = jnp.concatenate([dst, epad]).reshape(NS, NBT, BATCH)
    dst32 = jnp.concatenate([dst, epad]).reshape(32, EPAD // 32 // BATCH, BATCH)
    u32 = jnp.concatenate([u, epad]).reshape(32, EPAD // 32 // BDEC, BDEC)
    v32 = jnp.concatenate([v, epad]).reshape(32, EPAD // 32 // BDEC, BDEC)
    ea_pad = jnp.zeros((EPAD, D_EDGE), f32).at[:N_EDGES].set(edge_attr)
    zeros_n = jnp.zeros((NP, 128), f32)
    zeros_1 = jnp.zeros((NP,), f32)
    wu = mlp_w1[:HID]
    wv = mlp_w1[HID:2 * HID]
    we = mlp_w1[2 * HID:]
    b1r = b1.reshape(1, HID)
    b2r = b2.reshape(1, HID)
    mb1 = mlp_b1.reshape(1, HID)
    mb2 = mlp_b2.reshape(1, 1)

    deg = _deg_kernel(dst32, zeros_1)

    bm = 256
    grid = (NP // bm,)
    y0, y1 = pl.pallas_call(
        _k1_body,
        grid=grid,
        in_specs=[
            pl.BlockSpec((bm, D_IN), lambda i: (i, 0)),
            pl.BlockSpec((D_IN, HID), lambda i: (0, 0)),
            pl.BlockSpec((NC, bm), lambda i: (0, i)),
        ],
        out_specs=[
            pl.BlockSpec((bm, 128), lambda i: (i, 0)),
            pl.BlockSpec((bm, 128), lambda i: (i, 0)),
        ],
        out_shape=[
            jax.ShapeDtypeStruct((NP, 128), f32),
            jax.ShapeDtypeStruct((NP, 128), f32),
        ],
    )(x_pad, W1, deg)

    s0, s1 = _conv_kernel(y0, y1, src16, dst16, zeros_n)

    y20, y21 = pl.pallas_call(
        _k2_body,
        grid=grid,
        in_specs=[
            pl.BlockSpec((bm, 128), lambda i: (i, 0)),
            pl.BlockSpec((bm, 128), lambda i: (i, 0)),
            pl.BlockSpec((bm, 128), lambda i: (i, 0)),
            pl.BlockSpec((bm, 128), lambda i: (i, 0)),
            pl.BlockSpec((NC, bm), lambda i: (0, i)),
            pl.BlockSpec((1, HID), lambda i: (0, 0)),
            pl.BlockSpec((HID, HID), lambda i: (0, 0)),
        ],
        out_specs=[
            pl.BlockSpec((bm, 128), lambda i: (i, 0)),
            pl.BlockSpec((bm, 128), lambda i: (i, 0)),
        ],
        out_shape=[
            jax.ShapeDtypeStruct((NP, 128), f32),
            jax.ShapeDtypeStruct((NP, 128), f32),
        ],
    )(s0, s1, y0, y1, deg, b1r, W2)

    t0, t1 = _conv_kernel(y20, y21, src16, dst16, zeros_n)

    A, B = pl.pallas_call(
        _k3_body,
        grid=grid,
        in_specs=[
            pl.BlockSpec((bm, 128), lambda i: (i, 0)),
            pl.BlockSpec((bm, 128), lambda i: (i, 0)),
            pl.BlockSpec((bm, 128), lambda i: (i, 0)),
            pl.BlockSpec((bm, 128), lambda i: (i, 0)),
            pl.BlockSpec((NC, bm), lambda i: (0, i)),
            pl.BlockSpec((1, HID), lambda i: (0, 0)),
            pl.BlockSpec((HID, HID), lambda i: (0, 0)),
            pl.BlockSpec((HID, HID), lambda i: (0, 0)),
        ],
        out_specs=[
            pl.BlockSpec((bm, 128), lambda i: (i, 0)),
            pl.BlockSpec((bm, 128), lambda i: (i, 0)),
        ],
        out_shape=[
            jax.ShapeDtypeStruct((NP, 128), jnp.int32),
            jax.ShapeDtypeStruct((NP, 128), jnp.int32),
        ],
    )(t0, t1, y20, y21, deg, b2r, wu, wv)

    GA, GB = _decode_kernel(A, B, u32, v32)

    bm4 = 2048
    out = pl.pallas_call(
        _k4_body,
        grid=(EPAD // bm4,),
        in_specs=[
            pl.BlockSpec((bm4, 128), lambda i: (i, 0)),
            pl.BlockSpec((bm4, 128), lambda i: (i, 0)),
            pl.BlockSpec((bm4, D_EDGE), lambda i: (i, 0)),
            pl.BlockSpec((D_EDGE, HID), lambda i: (0, 0)),
            pl.BlockSpec((1, HID), lambda i: (0, 0)),
            pl.BlockSpec((HID, 1), lambda i: (0, 0)),
            pl.BlockSpec((1, 1), lambda i: (0, 0)),
        ],
        out_specs=pl.BlockSpec((bm4, 1), lambda i: (i, 0)),
        out_shape=jax.ShapeDtypeStruct((EPAD, 1), f32),
    )(GA, GB, ea_pad, we, mb1, mlp_w2, mb2)

    return out[:N_EDGES].reshape(-1)


# speedup vs baseline: 5.6350x; 1.0591x over previous
import functools

import jax
import jax.numpy as jnp
from jax import lax
from jax.experimental import pallas as pl
from jax.experimental.pallas import tpu as pltpu
from jax.experimental.pallas import tpu_sc as plsc

N_NODES = 10000
N_EDGES = 160000
D_IN = 256
HID = 256
D_EDGE = 16

NP = 10240
SENT = 10200
EPAD = 163840
BATCH = 128
NC = 2
NS = 16
NBT = EPAD // NS // BATCH
NPH = NBT // 2

_mesh = plsc.VectorSubcoreMesh(core_axis_name="c", subcore_axis_name="s")


@functools.partial(
    pl.kernel,
    out_type=jax.ShapeDtypeStruct((NC, NP), jnp.float32),
    mesh=_mesh,
    scratch_types=[
        pltpu.VMEM((EPAD // 32 // BATCH, BATCH), jnp.int32),
        pltpu.VMEM((BATCH,), jnp.float32),
        pltpu.VMEM_SHARED((NP,), jnp.float32),
        pltpu.SemaphoreType.DMA,
    ],
)
def _deg_kernel(dst_hbm, zeros_hbm, out_hbm, idx_v, ones_v, acc, sem):
    c = lax.axis_index("c")
    s = lax.axis_index("s")
    wid = s * NC + c
    nrow = NP // NS
    for i in range(BATCH // 16):
        ones_v[pl.ds(i * 16, 16)] = jnp.ones((16,), jnp.float32)
    pltpu.sync_copy(zeros_hbm.at[pl.ds(s * nrow, nrow)], acc.at[pl.ds(s * nrow, nrow)])
    pltpu.sync_copy(dst_hbm.at[wid], idx_v)
    plsc.subcore_barrier()

    def body(j, carry):
        pltpu.sync_copy(ones_v, acc.at[idx_v.at[j]], add=True)
        return carry

    lax.fori_loop(0, EPAD // 32 // BATCH, body, 0)
    plsc.subcore_barrier()
    pltpu.sync_copy(acc.at[pl.ds(s * nrow, nrow)], out_hbm.at[c, pl.ds(s * nrow, nrow)])


@functools.partial(
    pl.kernel,
    out_type=(
        jax.ShapeDtypeStruct((NP, 128), jnp.float32),
        jax.ShapeDtypeStruct((NP, 128), jnp.float32),
    ),
    mesh=_mesh,
    scratch_types=[
        pltpu.VMEM((NPH, BATCH), jnp.int32),
        pltpu.VMEM((NPH, BATCH), jnp.int32),
        pltpu.VMEM((BATCH, 128), jnp.float32),
        pltpu.VMEM((BATCH, 128), jnp.float32),
        pltpu.VMEM_SHARED((NP, 128), jnp.float32),
        pltpu.SemaphoreType.DMA,
        pltpu.SemaphoreType.DMA,
    ],
)
def _conv_kernel(y0_hbm, y1_hbm, src_hbm, dst_hbm, zeros_hbm, out0_hbm, out1_hbm,
                 src_v, dst_v, buf0, buf1, acc, sem0, sem1):
    c = lax.axis_index("c")
    s = lax.axis_index("s")
    nrow = NP // NS
    UN = 8
    pltpu.sync_copy(zeros_hbm.at[pl.ds(s * nrow, nrow)], acc.at[pl.ds(s * nrow, nrow)])
    plsc.subcore_barrier()

    def run(table_hbm, out_hbm):
        bufs = (buf0, buf1)
        sems = (sem0, sem1)

        def issue(j, b):
            pltpu.async_copy(table_hbm.at[src_v.at[j]], bufs[b], sems[b])

        def wait(b):
            pltpu.make_async_copy(table_hbm.at[pl.ds(0, BATCH)], bufs[b], sems[b]).wait()

        for ph in range(2):
            pltpu.sync_copy(src_hbm.at[s, pl.ds(ph * NPH, NPH)], src_v)
            pltpu.sync_copy(dst_hbm.at[s, pl.ds(ph * NPH, NPH)], dst_v)
            issue(0, 0)

            def body(i, carry):
                base = i * UN
                for k in range(UN):
                    cur = k & 1
                    nxt = 1 - cur
                    nj = base + k + 1

                    @pl.when(nj < NPH)
                    def _():
                        issue(nj, nxt)

                    wait(cur)
                    pltpu.sync_copy(bufs[cur], acc.at[dst_v.at[base + k]], add=True)
                return carry

            lax.fori_loop(0, NPH // UN, body, 0)
        plsc.subcore_barrier()
        pltpu.sync_copy(acc.at[pl.ds(s * nrow, nrow)], out_hbm.at[pl.ds(s * nrow, nrow)])

    @pl.when(c == 0)
    def _():
        run(y0_hbm, out0_hbm)

    @pl.when(c == 1)
    def _():
        run(y1_hbm, out1_hbm)


BDEC = 128


NBD = 20


@functools.partial(
    pl.kernel,
    out_type=(
        jax.ShapeDtypeStruct((EPAD // 2, 128), jnp.int32),
        jax.ShapeDtypeStruct((EPAD // 2, 128), jnp.int32),
    ),
    mesh=_mesh,
    scratch_types=[
        pltpu.VMEM((NBD, BDEC), jnp.int32),
        pltpu.VMEM((NBD, BDEC), jnp.int32),
        pltpu.VMEM((BDEC, 128), jnp.int32),
        pltpu.VMEM((BDEC, 128), jnp.int32),
        pltpu.VMEM((BDEC, 128), jnp.int32),
        pltpu.VMEM((BDEC, 128), jnp.int32),
        pltpu.SemaphoreType.DMA,
        pltpu.SemaphoreType.DMA,
        pltpu.SemaphoreType.DMA,
        pltpu.SemaphoreType.DMA,
        pltpu.SemaphoreType.DMA,
        pltpu.SemaphoreType.DMA,
        pltpu.SemaphoreType.DMA,
        pltpu.SemaphoreType.DMA,
    ],
)
def _decode_kernel(a_hbm, b_hbm, u_hbm, v_hbm, ga_hbm, gb_hbm,
                   u_v, v_v, a0, a1, b0, b1,
                   sa0, sa1, sb0, sb1, oa0, oa1, ob0, ob1):
    c = lax.axis_index("c")
    s = lax.axis_index("s")
    wid = s * NC + c
    nb = NBD
    UN = 4
    abufs = (a0, a1)
    bbufs = (b0, b1)
    asems = (sa0, sa1)
    bsems = (sb0, sb1)
    oasems = (oa0, oa1)
    obsems = (ob0, ob1)
    pltpu.sync_copy(u_hbm.at[wid], u_v)
    pltpu.sync_copy(v_hbm.at[wid], v_v)

    def issue(j, b):
        pltpu.async_copy(a_hbm.at[u_v.at[j]], abufs[b], asems[b])
        pltpu.async_copy(b_hbm.at[v_v.at[j]], bbufs[b], bsems[b])

    def wait(b):
        pltpu.make_async_copy(a_hbm.at[pl.ds(0, BDEC)], abufs[b], asems[b]).wait()
        pltpu.make_async_copy(b_hbm.at[pl.ds(0, BDEC)], bbufs[b], bsems[b]).wait()

    def drain_outs(b):
        pltpu.make_async_copy(a_hbm.at[pl.ds(0, BDEC)], abufs[b], oasems[b]).wait()
        pltpu.make_async_copy(a_hbm.at[pl.ds(0, BDEC)], bbufs[b], obsems[b]).wait()

    issue(0, 0)

    def body(i, carry):
        base = i * UN
        for k in range(UN):
            cur = k & 1
            nxt = 1 - cur
            nj = base + k + 1

            @pl.when((nj < nb) & (nj >= 2))
            def _():
                drain_outs(nxt)

            @pl.when(nj < nb)
            def _():
                issue(nj, nxt)

            wait(cur)
            off = wid * (nb * BDEC) + (base + k) * BDEC
            pltpu.async_copy(abufs[cur], ga_hbm.at[pl.ds(off, BDEC)], oasems[cur])
            pltpu.async_copy(bbufs[cur], gb_hbm.at[pl.ds(off, BDEC)], obsems[cur])
        return carry

    lax.fori_loop(0, nb // UN, body, 0)
    drain_outs(0)
    drain_outs(1)


def _k1_body(x_ref, w_ref, deg_ref, y0_ref, y1_ref):
    d = deg_ref[...]
    dis = lax.rsqrt(d[0] + d[1] + 1.0)
    yw = dis[:, None] * jnp.dot(x_ref[...], w_ref[...],
                                preferred_element_type=jnp.float32)
    y0_ref[...] = yw[:, :128]
    y1_ref[...] = yw[:, 128:]


def _k2_body(s0_ref, s1_ref, y0_ref, y1_ref, deg_ref, b1_ref, w2_ref,
             o0_ref, o1_ref):
    d = deg_ref[...]
    dis = lax.rsqrt(d[0] + d[1] + 1.0)
    sfull = jnp.concatenate([s0_ref[...] + y0_ref[...],
                             s1_ref[...] + y1_ref[...]], axis=1)
    h = jnp.maximum(dis[:, None] * sfull + b1_ref[...], 0.0)
    hw2 = dis[:, None] * jnp.dot(h, w2_ref[...],
                                 preferred_element_type=jnp.float32)
    o0_ref[...] = hw2[:, :128]
    o1_ref[...] = hw2[:, 128:]


def _k3_body(s0_ref, s1_ref, y0_ref, y1_ref, deg_ref, b2_ref, wu_ref, wv_ref,
             a_ref, b_ref):
    d = deg_ref[...]
    dis = lax.rsqrt(d[0] + d[1] + 1.0)
    sfull = jnp.concatenate([s0_ref[...] + y0_ref[...],
                             s1_ref[...] + y1_ref[...]], axis=1)
    z = dis[:, None] * sfull + b2_ref[...]
    A = jnp.dot(z, wu_ref[...], preferred_element_type=jnp.float32)
    B = jnp.dot(z, wv_ref[...], preferred_element_type=jnp.float32)

    def pack(m):
        lo = jax.lax.bitcast_convert_type(m[:, :128].astype(jnp.bfloat16),
                                          jnp.uint16).astype(jnp.uint32)
        hi = jax.lax.bitcast_convert_type(m[:, 128:].astype(jnp.bfloat16),
                                          jnp.uint16).astype(jnp.uint32)
        return (lo | (hi << 16)).astype(jnp.int32)

    a_ref[...] = pack(A)
    b_ref[...] = pack(B)


def _unpack(gw):
    lo = jax.lax.bitcast_convert_type(
        (gw & 0xFFFF).astype(jnp.uint16), jnp.bfloat16)
    hi = jax.lax.bitcast_convert_type(
        ((gw >> 16) & 0xFFFF).astype(jnp.uint16), jnp.bfloat16)
    return jnp.concatenate([lo, hi], axis=1).astype(jnp.float32)


def _k4_body(ga_ref, gb_ref, ea_ref, we_ref, mb1_ref, w2_ref, mb2_ref, o_ref):
    g = _unpack(ga_ref[...]) + _unpack(gb_ref[...])
    hdn = jnp.maximum(
        g
        + jnp.dot(ea_ref[...], we_ref[...], preferred_element_type=jnp.float32)
        + mb1_ref[...], 0.0)
    o_ref[...] = jnp.dot(hdn, w2_ref[...],
                         preferred_element_type=jnp.float32) + mb2_ref[...]


def kernel(x, edge_index, edge_label_index, edge_attr,
           W1, b1, W2, b2, mlp_w1, mlp_b1, mlp_w2, mlp_b2):
    f32 = jnp.float32
    x_pad = jnp.zeros((NP, D_IN), f32).at[:N_NODES].set(x)
    src = edge_index[0].astype(jnp.int32)
    dst = edge_index[1].astype(jnp.int32)
    u = edge_label_index[0].astype(jnp.int32)
    v = edge_label_index[1].astype(jnp.int32)
    epad = jnp.full((EPAD - N_EDGES,), SENT, jnp.int32)
    src16 = jnp.concatenate([src, epad]).reshape(NS, NBT, BATCH)
    dst16 = jnp.concatenate([dst, epad]).reshape(NS, NBT, BATCH)
    dst32 = jnp.concatenate([dst, epad]).reshape(32, EPAD // 32 // BATCH, BATCH)
    u32 = jnp.concatenate([u, epad]).reshape(32, 2 * NBD, BDEC)
    v32 = jnp.concatenate([v, epad]).reshape(32, 2 * NBD, BDEC)
    u32a, u32b = u32[:, :NBD], u32[:, NBD:]
    v32a, v32b = v32[:, :NBD], v32[:, NBD:]
    ea_pad = jnp.zeros((EPAD, D_EDGE), f32).at[:N_EDGES].set(edge_attr)
    zeros_n = jnp.zeros((NP, 128), f32)
    zeros_1 = jnp.zeros((NP,), f32)
    wu = mlp_w1[:HID]
    wv = mlp_w1[HID:2 * HID]
    we = mlp_w1[2 * HID:]
    b1r = b1.reshape(1, HID)
    b2r = b2.reshape(1, HID)
    mb1 = mlp_b1.reshape(1, HID)
    mb2 = mlp_b2.reshape(1, 1)

    deg = _deg_kernel(dst32, zeros_1)

    bm = 256
    grid = (NP // bm,)
    y0, y1 = pl.pallas_call(
        _k1_body,
        grid=grid,
        in_specs=[
            pl.BlockSpec((bm, D_IN), lambda i: (i, 0)),
            pl.BlockSpec((D_IN, HID), lambda i: (0, 0)),
            pl.BlockSpec((NC, bm), lambda i: (0, i)),
        ],
        out_specs=[
            pl.BlockSpec((bm, 128), lambda i: (i, 0)),
            pl.BlockSpec((bm, 128), lambda i: (i, 0)),
        ],
        out_shape=[
            jax.ShapeDtypeStruct((NP, 128), f32),
            jax.ShapeDtypeStruct((NP, 128), f32),
        ],
    )(x_pad, W1, deg)

    s0, s1 = _conv_kernel(y0, y1, src16, dst16, zeros_n)

    y20, y21 = pl.pallas_call(
        _k2_body,
        grid=grid,
        in_specs=[
            pl.BlockSpec((bm, 128), lambda i: (i, 0)),
            pl.BlockSpec((bm, 128), lambda i: (i, 0)),
            pl.BlockSpec((bm, 128), lambda i: (i, 0)),
            pl.BlockSpec((bm, 128), lambda i: (i, 0)),
            pl.BlockSpec((NC, bm), lambda i: (0, i)),
            pl.BlockSpec((1, HID), lambda i: (0, 0)),
            pl.BlockSpec((HID, HID), lambda i: (0, 0)),
        ],
        out_specs=[
            pl.BlockSpec((bm, 128), lambda i: (i, 0)),
            pl.BlockSpec((bm, 128), lambda i: (i, 0)),
        ],
        out_shape=[
            jax.ShapeDtypeStruct((NP, 128), f32),
            jax.ShapeDtypeStruct((NP, 128), f32),
        ],
    )(s0, s1, y0, y1, deg, b1r, W2)

    t0, t1 = _conv_kernel(y20, y21, src16, dst16, zeros_n)

    A, B = pl.pallas_call(
        _k3_body,
        grid=grid,
        in_specs=[
            pl.BlockSpec((bm, 128), lambda i: (i, 0)),
            pl.BlockSpec((bm, 128), lambda i: (i, 0)),
            pl.BlockSpec((bm, 128), lambda i: (i, 0)),
            pl.BlockSpec((bm, 128), lambda i: (i, 0)),
            pl.BlockSpec((NC, bm), lambda i: (0, i)),
            pl.BlockSpec((1, HID), lambda i: (0, 0)),
            pl.BlockSpec((HID, HID), lambda i: (0, 0)),
            pl.BlockSpec((HID, HID), lambda i: (0, 0)),
        ],
        out_specs=[
            pl.BlockSpec((bm, 128), lambda i: (i, 0)),
            pl.BlockSpec((bm, 128), lambda i: (i, 0)),
        ],
        out_shape=[
            jax.ShapeDtypeStruct((NP, 128), jnp.int32),
            jax.ShapeDtypeStruct((NP, 128), jnp.int32),
        ],
    )(t0, t1, y20, y21, deg, b2r, wu, wv)

    GAa, GBa = _decode_kernel(A, B, u32a, v32a)
    GAb, GBb = _decode_kernel(A, B, u32b, v32b)

    half = EPAD // 2
    per = half // 32
    ea_w = ea_pad.reshape(32, 2, per, D_EDGE)
    ea_a = ea_w[:, 0].reshape(half, D_EDGE)
    ea_b = ea_w[:, 1].reshape(half, D_EDGE)
    bm4 = 2560
    k4 = functools.partial(
        pl.pallas_call,
        _k4_body,
        grid=(half // bm4,),
        in_specs=[
            pl.BlockSpec((bm4, 128), lambda i: (i, 0)),
            pl.BlockSpec((bm4, 128), lambda i: (i, 0)),
            pl.BlockSpec((bm4, D_EDGE), lambda i: (i, 0)),
            pl.BlockSpec((D_EDGE, HID), lambda i: (0, 0)),
            pl.BlockSpec((1, HID), lambda i: (0, 0)),
            pl.BlockSpec((HID, 1), lambda i: (0, 0)),
            pl.BlockSpec((1, 1), lambda i: (0, 0)),
        ],
        out_specs=pl.BlockSpec((bm4, 1), lambda i: (i, 0)),
        out_shape=jax.ShapeDtypeStruct((half, 1), f32),
    )
    outa = k4()(GAa, GBa, ea_a, we, mb1, mlp_w2, mb2)
    outb = k4()(GAb, GBb, ea_b, we, mb1, mlp_w2, mb2)

    out = jnp.concatenate([outa.reshape(32, per), outb.reshape(32, per)],
                          axis=1).reshape(-1)
    return out[:N_EDGES]


# speedup vs baseline: 5.6365x; 1.0003x over previous
import functools

import jax
import jax.numpy as jnp
from jax import lax
from jax.experimental import pallas as pl
from jax.experimental.pallas import tpu as pltpu
from jax.experimental.pallas import tpu_sc as plsc

N_NODES = 10000
N_EDGES = 160000
D_IN = 256
HID = 256
D_EDGE = 16

NP = 10240
SENT = 10200
EPAD = 163840
BATCH = 128
NC = 2
NS = 16
NBT = EPAD // NS // BATCH
NPH = NBT // 2

_mesh = plsc.VectorSubcoreMesh(core_axis_name="c", subcore_axis_name="s")


@functools.partial(
    pl.kernel,
    out_type=jax.ShapeDtypeStruct((NC, NP), jnp.float32),
    mesh=_mesh,
    scratch_types=[
        pltpu.VMEM((EPAD // 32 // BATCH, BATCH), jnp.int32),
        pltpu.VMEM((BATCH,), jnp.float32),
        pltpu.VMEM_SHARED((NP,), jnp.float32),
        pltpu.SemaphoreType.DMA,
    ],
)
def _deg_kernel(dst_hbm, zeros_hbm, out_hbm, idx_v, ones_v, acc, sem):
    c = lax.axis_index("c")
    s = lax.axis_index("s")
    wid = s * NC + c
    nrow = NP // NS
    for i in range(BATCH // 16):
        ones_v[pl.ds(i * 16, 16)] = jnp.ones((16,), jnp.float32)
    pltpu.sync_copy(zeros_hbm.at[pl.ds(s * nrow, nrow)], acc.at[pl.ds(s * nrow, nrow)])
    pltpu.sync_copy(dst_hbm.at[wid], idx_v)
    plsc.subcore_barrier()

    def body(j, carry):
        pltpu.sync_copy(ones_v, acc.at[idx_v.at[j]], add=True)
        return carry

    lax.fori_loop(0, EPAD // 32 // BATCH, body, 0)
    plsc.subcore_barrier()
    pltpu.sync_copy(acc.at[pl.ds(s * nrow, nrow)], out_hbm.at[c, pl.ds(s * nrow, nrow)])


@functools.partial(
    pl.kernel,
    out_type=(
        jax.ShapeDtypeStruct((NP, 128), jnp.float32),
        jax.ShapeDtypeStruct((NP, 128), jnp.float32),
    ),
    mesh=_mesh,
    scratch_types=[
        pltpu.VMEM((NPH, BATCH), jnp.int32),
        pltpu.VMEM((NPH, BATCH), jnp.int32),
        pltpu.VMEM((BATCH, 128), jnp.float32),
        pltpu.VMEM((BATCH, 128), jnp.float32),
        pltpu.VMEM_SHARED((NP, 128), jnp.float32),
        pltpu.SemaphoreType.DMA,
        pltpu.SemaphoreType.DMA,
        pltpu.SemaphoreType.DMA,
        pltpu.SemaphoreType.DMA,
    ],
)
def _conv_kernel(y0_hbm, y1_hbm, src_hbm, dst_hbm, zeros_hbm, out0_hbm, out1_hbm,
                 src_v, dst_v, buf0, buf1, acc, sem0, sem1, ssem0, ssem1):
    c = lax.axis_index("c")
    s = lax.axis_index("s")
    nrow = NP // NS
    UN = 8
    pltpu.sync_copy(zeros_hbm.at[pl.ds(s * nrow, nrow)], acc.at[pl.ds(s * nrow, nrow)])
    plsc.subcore_barrier()

    def run(table_hbm, out_hbm):
        bufs = (buf0, buf1)
        sems = (sem0, sem1)
        ssems = (ssem0, ssem1)

        def issue(j, b):
            pltpu.async_copy(table_hbm.at[src_v.at[j]], bufs[b], sems[b])

        def wait(b):
            pltpu.make_async_copy(table_hbm.at[pl.ds(0, BATCH)], bufs[b], sems[b]).wait()

        def drain_scatter(b):
            pltpu.make_async_copy(bufs[b], acc.at[pl.ds(0, BATCH)], ssems[b]).wait()

        for ph in range(2):
            pltpu.sync_copy(src_hbm.at[s, pl.ds(ph * NPH, NPH)], src_v)
            pltpu.sync_copy(dst_hbm.at[s, pl.ds(ph * NPH, NPH)], dst_v)
            issue(0, 0)

            def body(i, carry):
                base = i * UN
                for k in range(UN):
                    cur = k & 1
                    nxt = 1 - cur
                    nj = base + k + 1

                    @pl.when((nj < NPH) & (nj >= 2))
                    def _():
                        drain_scatter(nxt)

                    @pl.when(nj < NPH)
                    def _():
                        issue(nj, nxt)

                    wait(cur)
                    pltpu.async_copy(bufs[cur], acc.at[dst_v.at[base + k]],
                                     ssems[cur], add=True)
                return carry

            lax.fori_loop(0, NPH // UN, body, 0)
            drain_scatter(0)
            drain_scatter(1)
        plsc.subcore_barrier()
        pltpu.sync_copy(acc.at[pl.ds(s * nrow, nrow)], out_hbm.at[pl.ds(s * nrow, nrow)])

    @pl.when(c == 0)
    def _():
        run(y0_hbm, out0_hbm)

    @pl.when(c == 1)
    def _():
        run(y1_hbm, out1_hbm)


BDEC = 128


NBD = 20


@functools.partial(
    pl.kernel,
    out_type=(
        jax.ShapeDtypeStruct((EPAD // 2, 128), jnp.int32),
        jax.ShapeDtypeStruct((EPAD // 2, 128), jnp.int32),
    ),
    mesh=_mesh,
    scratch_types=[
        pltpu.VMEM((NBD, BDEC), jnp.int32),
        pltpu.VMEM((NBD, BDEC), jnp.int32),
        pltpu.VMEM((BDEC, 128), jnp.int32),
        pltpu.VMEM((BDEC, 128), jnp.int32),
        pltpu.VMEM((BDEC, 128), jnp.int32),
        pltpu.VMEM((BDEC, 128), jnp.int32),
        pltpu.SemaphoreType.DMA,
        pltpu.SemaphoreType.DMA,
        pltpu.SemaphoreType.DMA,
        pltpu.SemaphoreType.DMA,
        pltpu.SemaphoreType.DMA,
        pltpu.SemaphoreType.DMA,
        pltpu.SemaphoreType.DMA,
        pltpu.SemaphoreType.DMA,
    ],
)
def _decode_kernel(a_hbm, b_hbm, u_hbm, v_hbm, ga_hbm, gb_hbm,
                   u_v, v_v, a0, a1, b0, b1,
                   sa0, sa1, sb0, sb1, oa0, oa1, ob0, ob1):
    c = lax.axis_index("c")
    s = lax.axis_index("s")
    wid = s * NC + c
    nb = NBD
    UN = 4
    abufs = (a0, a1)
    bbufs = (b0, b1)
    asems = (sa0, sa1)
    bsems = (sb0, sb1)
    oasems = (oa0, oa1)
    obsems = (ob0, ob1)
    pltpu.sync_copy(u_hbm.at[wid], u_v)
    pltpu.sync_copy(v_hbm.at[wid], v_v)

    def issue(j, b):
        pltpu.async_copy(a_hbm.at[u_v.at[j]], abufs[b], asems[b])
        pltpu.async_copy(b_hbm.at[v_v.at[j]], bbufs[b], bsems[b])

    def wait(b):
        pltpu.make_async_copy(a_hbm.at[pl.ds(0, BDEC)], abufs[b], asems[b]).wait()
        pltpu.make_async_copy(b_hbm.at[pl.ds(0, BDEC)], bbufs[b], bsems[b]).wait()

    def drain_outs(b):
        pltpu.make_async_copy(a_hbm.at[pl.ds(0, BDEC)], abufs[b], oasems[b]).wait()
        pltpu.make_async_copy(a_hbm.at[pl.ds(0, BDEC)], bbufs[b], obsems[b]).wait()

    issue(0, 0)

    def body(i, carry):
        base = i * UN
        for k in range(UN):
            cur = k & 1
            nxt = 1 - cur
            nj = base + k + 1

            @pl.when((nj < nb) & (nj >= 2))
            def _():
                drain_outs(nxt)

            @pl.when(nj < nb)
            def _():
                issue(nj, nxt)

            wait(cur)
            off = wid * (nb * BDEC) + (base + k) * BDEC
            pltpu.async_copy(abufs[cur], ga_hbm.at[pl.ds(off, BDEC)], oasems[cur])
            pltpu.async_copy(bbufs[cur], gb_hbm.at[pl.ds(off, BDEC)], obsems[cur])
        return carry

    lax.fori_loop(0, nb // UN, body, 0)
    drain_outs(0)
    drain_outs(1)


def _k1_body(x_ref, w_ref, deg_ref, y0_ref, y1_ref):
    d = deg_ref[...]
    dis = lax.rsqrt(d[0] + d[1] + 1.0)
    yw = dis[:, None] * jnp.dot(x_ref[...], w_ref[...],
                                preferred_element_type=jnp.float32)
    y0_ref[...] = yw[:, :128]
    y1_ref[...] = yw[:, 128:]


def _k2_body(s0_ref, s1_ref, y0_ref, y1_ref, deg_ref, b1_ref, w2_ref,
             o0_ref, o1_ref):
    d = deg_ref[...]
    dis = lax.rsqrt(d[0] + d[1] + 1.0)
    sfull = jnp.concatenate([s0_ref[...] + y0_ref[...],
                             s1_ref[...] + y1_ref[...]], axis=1)
    h = jnp.maximum(dis[:, None] * sfull + b1_ref[...], 0.0)
    hw2 = dis[:, None] * jnp.dot(h, w2_ref[...],
                                 preferred_element_type=jnp.float32)
    o0_ref[...] = hw2[:, :128]
    o1_ref[...] = hw2[:, 128:]


def _k3_body(s0_ref, s1_ref, y0_ref, y1_ref, deg_ref, b2_ref, wu_ref, wv_ref,
             a_ref, b_ref):
    d = deg_ref[...]
    dis = lax.rsqrt(d[0] + d[1] + 1.0)
    sfull = jnp.concatenate([s0_ref[...] + y0_ref[...],
                             s1_ref[...] + y1_ref[...]], axis=1)
    z = dis[:, None] * sfull + b2_ref[...]
    A = jnp.dot(z, wu_ref[...], preferred_element_type=jnp.float32)
    B = jnp.dot(z, wv_ref[...], preferred_element_type=jnp.float32)

    def pack(m):
        lo = jax.lax.bitcast_convert_type(m[:, :128].astype(jnp.bfloat16),
                                          jnp.uint16).astype(jnp.uint32)
        hi = jax.lax.bitcast_convert_type(m[:, 128:].astype(jnp.bfloat16),
                                          jnp.uint16).astype(jnp.uint32)
        return (lo | (hi << 16)).astype(jnp.int32)

    a_ref[...] = pack(A)
    b_ref[...] = pack(B)


def _unpack(gw):
    lo = jax.lax.bitcast_convert_type(
        (gw & 0xFFFF).astype(jnp.uint16), jnp.bfloat16)
    hi = jax.lax.bitcast_convert_type(
        ((gw >> 16) & 0xFFFF).astype(jnp.uint16), jnp.bfloat16)
    return jnp.concatenate([lo, hi], axis=1).astype(jnp.float32)


def _k4_body(ga_ref, gb_ref, ea_ref, we_ref, mb1_ref, w2_ref, mb2_ref, o_ref):
    g = _unpack(ga_ref[...]) + _unpack(gb_ref[...])
    hdn = jnp.maximum(
        g
        + jnp.dot(ea_ref[...], we_ref[...], preferred_element_type=jnp.float32)
        + mb1_ref[...], 0.0)
    o_ref[...] = jnp.dot(hdn, w2_ref[...],
                         preferred_element_type=jnp.float32) + mb2_ref[...]


def kernel(x, edge_index, edge_label_index, edge_attr,
           W1, b1, W2, b2, mlp_w1, mlp_b1, mlp_w2, mlp_b2):
    f32 = jnp.float32
    x_pad = jnp.zeros((NP, D_IN), f32).at[:N_NODES].set(x)
    src = edge_index[0].astype(jnp.int32)
    dst = edge_index[1].astype(jnp.int32)
    u = edge_label_index[0].astype(jnp.int32)
    v = edge_label_index[1].astype(jnp.int32)
    epad = jnp.full((EPAD - N_EDGES,), SENT, jnp.int32)
    src16 = jnp.concatenate([src, epad]).reshape(NS, NBT, BATCH)
    dst16 = jnp.concatenate([dst, epad]).reshape(NS, NBT, BATCH)
    dst32 = jnp.concatenate([dst, epad]).reshape(32, EPAD // 32 // BATCH, BATCH)
    u32 = jnp.concatenate([u, epad]).reshape(32, 2 * NBD, BDEC)
    v32 = jnp.concatenate([v, epad]).reshape(32, 2 * NBD, BDEC)
    u32a, u32b = u32[:, :NBD], u32[:, NBD:]
    v32a, v32b = v32[:, :NBD], v32[:, NBD:]
    ea_pad = jnp.zeros((EPAD, D_EDGE), f32).at[:N_EDGES].set(edge_attr)
    zeros_n = jnp.zeros((NP, 128), f32)
    zeros_1 = jnp.zeros((NP,), f32)
    wu = mlp_w1[:HID]
    wv = mlp_w1[HID:2 * HID]
    we = mlp_w1[2 * HID:]
    b1r = b1.reshape(1, HID)
    b2r = b2.reshape(1, HID)
    mb1 = mlp_b1.reshape(1, HID)
    mb2 = mlp_b2.reshape(1, 1)

    deg = _deg_kernel(dst32, zeros_1)

    bm = 256
    grid = (NP // bm,)
    y0, y1 = pl.pallas_call(
        _k1_body,
        grid=grid,
        in_specs=[
            pl.BlockSpec((bm, D_IN), lambda i: (i, 0)),
            pl.BlockSpec((D_IN, HID), lambda i: (0, 0)),
            pl.BlockSpec((NC, bm), lambda i: (0, i)),
        ],
        out_specs=[
            pl.BlockSpec((bm, 128), lambda i: (i, 0)),
            pl.BlockSpec((bm, 128), lambda i: (i, 0)),
        ],
        out_shape=[
            jax.ShapeDtypeStruct((NP, 128), f32),
            jax.ShapeDtypeStruct((NP, 128), f32),
        ],
    )(x_pad, W1, deg)

    s0, s1 = _conv_kernel(y0, y1, src16, dst16, zeros_n)

    y20, y21 = pl.pallas_call(
        _k2_body,
        grid=grid,
        in_specs=[
            pl.BlockSpec((bm, 128), lambda i: (i, 0)),
            pl.BlockSpec((bm, 128), lambda i: (i, 0)),
            pl.BlockSpec((bm, 128), lambda i: (i, 0)),
            pl.BlockSpec((bm, 128), lambda i: (i, 0)),
            pl.BlockSpec((NC, bm), lambda i: (0, i)),
            pl.BlockSpec((1, HID), lambda i: (0, 0)),
            pl.BlockSpec((HID, HID), lambda i: (0, 0)),
        ],
        out_specs=[
            pl.BlockSpec((bm, 128), lambda i: (i, 0)),
            pl.BlockSpec((bm, 128), lambda i: (i, 0)),
        ],
        out_shape=[
            jax.ShapeDtypeStruct((NP, 128), f32),
            jax.ShapeDtypeStruct((NP, 128), f32),
        ],
    )(s0, s1, y0, y1, deg, b1r, W2)

    t0, t1 = _conv_kernel(y20, y21, src16, dst16, zeros_n)

    A, B = pl.pallas_call(
        _k3_body,
        grid=grid,
        in_specs=[
            pl.BlockSpec((bm, 128), lambda i: (i, 0)),
            pl.BlockSpec((bm, 128), lambda i: (i, 0)),
            pl.BlockSpec((bm, 128), lambda i: (i, 0)),
            pl.BlockSpec((bm, 128), lambda i: (i, 0)),
            pl.BlockSpec((NC, bm), lambda i: (0, i)),
            pl.BlockSpec((1, HID), lambda i: (0, 0)),
            pl.BlockSpec((HID, HID), lambda i: (0, 0)),
            pl.BlockSpec((HID, HID), lambda i: (0, 0)),
        ],
        out_specs=[
            pl.BlockSpec((bm, 128), lambda i: (i, 0)),
            pl.BlockSpec((bm, 128), lambda i: (i, 0)),
        ],
        out_shape=[
            jax.ShapeDtypeStruct((NP, 128), jnp.int32),
            jax.ShapeDtypeStruct((NP, 128), jnp.int32),
        ],
    )(t0, t1, y20, y21, deg, b2r, wu, wv)

    GAa, GBa = _decode_kernel(A, B, u32a, v32a)
    GAb, GBb = _decode_kernel(A, B, u32b, v32b)

    half = EPAD // 2
    per = half // 32
    ea_w = ea_pad.reshape(32, 2, per, D_EDGE)
    ea_a = ea_w[:, 0].reshape(half, D_EDGE)
    ea_b = ea_w[:, 1].reshape(half, D_EDGE)
    bm4 = 2560
    k4 = functools.partial(
        pl.pallas_call,
        _k4_body,
        grid=(half // bm4,),
        in_specs=[
            pl.BlockSpec((bm4, 128), lambda i: (i, 0)),
            pl.BlockSpec((bm4, 128), lambda i: (i, 0)),
            pl.BlockSpec((bm4, D_EDGE), lambda i: (i, 0)),
            pl.BlockSpec((D_EDGE, HID), lambda i: (0, 0)),
            pl.BlockSpec((1, HID), lambda i: (0, 0)),
            pl.BlockSpec((HID, 1), lambda i: (0, 0)),
            pl.BlockSpec((1, 1), lambda i: (0, 0)),
        ],
        out_specs=pl.BlockSpec((bm4, 1), lambda i: (i, 0)),
        out_shape=jax.ShapeDtypeStruct((half, 1), f32),
    )
    outa = k4()(GAa, GBa, ea_a, we, mb1, mlp_w2, mb2)
    outb = k4()(GAb, GBb, ea_b, we, mb1, mlp_w2, mb2)

    out = jnp.concatenate([outa.reshape(32, per), outb.reshape(32, per)],
                          axis=1).reshape(-1)
    return out[:N_EDGES]
